# baseline (speedup 1.0000x reference)
import functools
import numpy as np
import jax
import jax.numpy as jnp
from jax import lax
from jax.experimental import pallas as pl
from jax.experimental.pallas import tpu as pltpu
from jax.experimental.pallas import tpu_sc as plsc

B = 16
N = 1024
NS = 256
L = 16
NCHUNK = N // L

_ROT = ((13, 15, 26, 6), (17, 29, 16, 24))
_PARITY = np.uint32(0x1BD11BDA)


def _threefry2x32(k1, k2, x0, x1):
    ks = (k1, k2, k1 ^ k2 ^ _PARITY)
    x0 = x0 + ks[0]
    x1 = x1 + ks[1]
    for i in range(5):
        for r in _ROT[i % 2]:
            x0 = x0 + x1
            x1 = (x1 << np.uint32(r)) | (x1 >> np.uint32(32 - r))
            x1 = x0 ^ x1
        x0 = x0 + ks[(i + 1) % 3]
        x1 = x1 + ks[(i + 2) % 3] + np.uint32(i + 1)
    return x0, x1


def _sampler_body(out_hbm, bits_v, row_v):
    cid = lax.axis_index("c")
    sid = lax.axis_index("s")
    w = sid * 2 + cid

    @pl.when(w < B)
    def _():
        zero_u = jnp.uint32(0)
        bw = lax.convert_element_type(w, jnp.uint32)
        bk1, bk2 = _threefry2x32(zero_u, jnp.uint32(42), zero_u, bw)
        sk1, sk2 = _threefry2x32(bk1, bk2, zero_u, jnp.uint32(1))

        iota_u = lax.iota(jnp.uint32, L)
        iota_i = lax.iota(jnp.int32, L)

        def tf_body(t, carry):
            base = t * (4 * L)
            for k in range(4):
                off = base + k * L
                x1 = iota_u + lax.convert_element_type(off, jnp.uint32)
                o0, o1 = _threefry2x32(sk1, sk2, jnp.zeros((L,), jnp.uint32), x1)
                bits_v[pl.ds(off, L)] = o0 ^ o1
            return carry

        lax.fori_loop(0, NCHUNK // 4, tf_body, jnp.int32(0))

        prefix = jnp.zeros((L,), jnp.uint32)
        for bit in range(31, -1, -1):
            cand = prefix | np.uint32(1 << bit)

            def cnt_body(t, cnt, cand=cand):
                base = t * (8 * L)
                for k in range(8):
                    v = bits_v[pl.ds(base + k * L, L)]
                    m = v < cand
                    cnt = cnt + plsc.all_reduce_population_count(m)
                return cnt

            cnt = lax.fori_loop(0, NCHUNK // 8, cnt_body,
                                jnp.zeros((L,), jnp.int32))
            prefix = jnp.where(cnt < NS, cand, prefix)
        thresh = prefix

        def c0_body(t, cnt):
            base = t * (8 * L)
            for k in range(8):
                m = bits_v[pl.ds(base + k * L, L)] < thresh
                cnt = cnt + plsc.all_reduce_population_count(m)
            return cnt

        c0 = lax.fori_loop(0, NCHUNK // 8, c0_body, jnp.zeros((L,), jnp.int32))
        need = NS - c0

        def cp_body(t, carry):
            off_vec, tie_vec = carry
            base = t * (4 * L)
            for k in range(4):
                o = base + k * L
                v = bits_v[pl.ds(o, L)]
                idxv = iota_i + o
                lt = v < thresh
                eq = v == thresh
                eq_i = eq.astype(jnp.int32)
                eq_excl = tie_vec + plsc.cumsum(eq_i) - eq_i
                sel = lt | (eq & (eq_excl < need))
                pos = off_vec + plsc.cumsum(sel.astype(jnp.int32)) - 1
                plsc.store_scatter(row_v, [pos], idxv, mask=sel)
                off_vec = off_vec + plsc.all_reduce_population_count(sel)
                tie_vec = tie_vec + plsc.all_reduce_population_count(eq)
            return off_vec, tie_vec

        lax.fori_loop(0, NCHUNK // 4, cp_body,
                      (jnp.zeros((L,), jnp.int32), jnp.zeros((L,), jnp.int32)))

        pltpu.sync_copy(row_v, out_hbm.at[w])


_sampler = functools.partial(
    pl.kernel,
    out_type=jax.ShapeDtypeStruct((B, NS), jnp.int32),
    mesh=plsc.VectorSubcoreMesh(core_axis_name="c", subcore_axis_name="s",
                                num_cores=2, num_subcores=16),
    scratch_types=[
        pltpu.VMEM((N,), jnp.uint32),
        pltpu.VMEM((NS,), jnp.int32),
    ],
    compiler_params=pltpu.CompilerParams(needs_layout_passes=False),
)(_sampler_body)


def kernel(images, features):
    del images, features
    return _sampler().astype(jnp.int64)
